# baseline (speedup 1.0000x reference)
import functools

import jax
import jax.numpy as jnp
from jax import lax
from jax.experimental import pallas as pl
from jax.experimental.pallas import tpu as pltpu
from jax.experimental.pallas import tpu_sc as plsc

S = 2048
D = 1024
F = 2816
E = 8
K = 2
T = 256
NPAIR = S * K
MAXTILES = NPAIR // T + (E - 1)
NP = MAXTILES * T

NC = 2
NS = 16
NW = NC * NS
TPW = S // NW
CHUNK = 32


def _router_body(x_ref, wr_ref, br_ref, pos_ref, w_ref, cnt_ref):
    x = x_ref[...]
    logits = jnp.dot(x, wr_ref[...], preferred_element_type=jnp.float32)
    logits = logits + br_ref[...]
    m = jnp.max(logits, axis=1, keepdims=True)
    ex = jnp.exp(logits - m)
    probs = ex / jnp.sum(ex, axis=1, keepdims=True)

    iota_e = lax.broadcasted_iota(jnp.int32, (S, E), 1)
    p1 = jnp.max(probs, axis=1, keepdims=True)
    i1 = jnp.min(jnp.where(probs == p1, iota_e, E), axis=1, keepdims=True)
    probs_m = jnp.where(iota_e == i1, -1.0, probs)
    p2 = jnp.max(probs_m, axis=1, keepdims=True)
    i2 = jnp.min(jnp.where(probs_m == p2, iota_e, E), axis=1, keepdims=True)
    ws = p1 + p2
    w0 = p1 / ws
    w1 = p2 / ws

    iota16 = lax.broadcasted_iota(jnp.int32, (S, 2 * E), 1)
    oh01 = jnp.where(iota16 < E, (i1 == iota16).astype(jnp.float32),
                     (i2 == (iota16 - E)).astype(jnp.float32))
    c = oh01
    sh = 1
    while sh < S:
        shifted = jnp.concatenate(
            [jnp.zeros((sh, 2 * E), jnp.float32), c[: S - sh, :]], axis=0)
        c = c + shifted
        sh *= 2
    tail = c[S - 1:S, :]
    cnt0 = tail[:, :E]
    cnt = cnt0 + tail[:, E:]
    tiles = jnp.floor((cnt + (T - 1)) * (1.0 / T))

    iota_e8 = lax.broadcasted_iota(jnp.int32, (S, E), 1)
    oh0 = (i1 == iota_e8).astype(jnp.float32)
    oh1 = (i2 == iota_e8).astype(jnp.float32)
    rank0 = jnp.sum(oh0 * c[:, :E], axis=1, keepdims=True) - 1.0
    rank1 = (jnp.sum(oh1 * c[:, E:], axis=1, keepdims=True) - 1.0
             + jnp.sum(oh1 * cnt0, axis=1, keepdims=True))
    seg0 = T * jnp.sum((iota_e8 < i1).astype(jnp.float32) * tiles,
                       axis=1, keepdims=True)
    seg1 = T * jnp.sum((iota_e8 < i2).astype(jnp.float32) * tiles,
                       axis=1, keepdims=True)
    pos_ref[:S, :] = (seg0 + rank0).astype(jnp.int32)
    pos_ref[S:, :] = (seg1 + rank1).astype(jnp.int32)
    wk = jnp.concatenate([w0, w1], axis=0)
    w_ref[...] = jnp.broadcast_to(wk, (NPAIR, 16))
    cnt_ref[...] = jnp.broadcast_to(cnt, (8, E))


def _run_router(x2, wr, br2):
    return pl.pallas_call(
        _router_body,
        out_shape=(
            jax.ShapeDtypeStruct((NPAIR, 1), jnp.int32),
            jax.ShapeDtypeStruct((NPAIR, 16), jnp.float32),
            jax.ShapeDtypeStruct((8, E), jnp.float32),
        ),
    )(x2, wr, br2)


def _dispatch_body(x_hbm, pos_hbm, xs_hbm, xbuf, idx0, idx1, sem):
    wid = lax.axis_index("s") * NC + lax.axis_index("c")
    base = wid * TPW
    pltpu.sync_copy(x_hbm.at[pl.ds(base, TPW)], xbuf)
    pltpu.sync_copy(pos_hbm.at[pl.ds(base, TPW)], idx0)
    pltpu.sync_copy(pos_hbm.at[pl.ds(S + base, TPW)], idx1)
    pltpu.async_copy(xbuf, xs_hbm.at[idx0], sem).wait()
    pltpu.async_copy(xbuf, xs_hbm.at[idx1], sem).wait()


@functools.cache
def _get_dispatch():
    return functools.partial(
        pl.kernel,
        out_type=jax.ShapeDtypeStruct((NP, D // 2), jnp.int32),
        mesh=plsc.VectorSubcoreMesh(
            core_axis_name="c", subcore_axis_name="s",
            num_cores=NC, num_subcores=NS),
        scratch_types=[
            pltpu.VMEM((TPW, D // 2), jnp.int32),
            pltpu.VMEM((TPW,), jnp.int32),
            pltpu.VMEM((TPW,), jnp.int32),
            pltpu.SemaphoreType.DMA,
        ],
    )(_dispatch_body)


def _ffn_body(te_ref, xs_ref, w1_ref, w3_ref, w2_ref, ys_ref):
    @pl.when(pl.program_id(0) < te_ref[MAXTILES])
    def _():
        xt = xs_ref[...]
        h1 = jnp.dot(xt, w1_ref[0], preferred_element_type=jnp.float32)
        h3 = jnp.dot(xt, w3_ref[0], preferred_element_type=jnp.float32)
        h = (h1 * jax.nn.sigmoid(h1) * h3).astype(jnp.bfloat16)
        ys_ref[...] = jnp.dot(h, w2_ref[0],
                              preferred_element_type=jnp.float32)


def _run_ffn(te, xs, w1, w3, w2):
    return pl.pallas_call(
        _ffn_body,
        grid_spec=pltpu.PrefetchScalarGridSpec(
            num_scalar_prefetch=1,
            grid=(MAXTILES,),
            in_specs=[
                pl.BlockSpec((T, D), lambda j, te: (j, 0)),
                pl.BlockSpec((1, D, F), lambda j, te: (te[j], 0, 0)),
                pl.BlockSpec((1, D, F), lambda j, te: (te[j], 0, 0)),
                pl.BlockSpec((1, F, D), lambda j, te: (te[j], 0, 0)),
            ],
            out_specs=pl.BlockSpec((T, D), lambda j, te: (j, 0)),
        ),
        out_shape=jax.ShapeDtypeStruct((NP, D), jnp.float32),
        name="moe_ffn",
    )(te, xs, w1, w3, w2)


def _combine_body(ys_hbm, pos_hbm, w_hbm, out_hbm,
                  buf0, buf1, obuf, idx0, idx1, wbuf0, wbuf1, sem):
    wid = lax.axis_index("s") * NC + lax.axis_index("c")
    for ci in range(TPW // CHUNK):
        base = wid * TPW + ci * CHUNK
        pltpu.sync_copy(pos_hbm.at[pl.ds(base, CHUNK)], idx0)
        pltpu.sync_copy(pos_hbm.at[pl.ds(S + base, CHUNK)], idx1)
        pltpu.sync_copy(w_hbm.at[pl.ds(base, CHUNK)], wbuf0)
        pltpu.sync_copy(w_hbm.at[pl.ds(S + base, CHUNK)], wbuf1)
        pltpu.async_copy(ys_hbm.at[idx0], buf0, sem).wait()
        pltpu.async_copy(ys_hbm.at[idx1], buf1, sem).wait()
        for i in range(CHUNK):
            wv0 = wbuf0[i, :]
            wv1 = wbuf1[i, :]

            def body(k, _, i=i, wv0=wv0, wv1=wv1):
                sl = pl.ds(k * 16, 16)
                obuf[i, sl] = wv0 * buf0[i, sl] + wv1 * buf1[i, sl]
                return 0

            lax.fori_loop(0, D // 16, body, 0, unroll=4)
        pltpu.sync_copy(obuf, out_hbm.at[pl.ds(base, CHUNK)])


@functools.cache
def _get_combine():
    return functools.partial(
        pl.kernel,
        out_type=jax.ShapeDtypeStruct((S, D), jnp.float32),
        mesh=plsc.VectorSubcoreMesh(
            core_axis_name="c", subcore_axis_name="s",
            num_cores=NC, num_subcores=NS),
        scratch_types=[
            pltpu.VMEM((CHUNK, D), jnp.float32),
            pltpu.VMEM((CHUNK, D), jnp.float32),
            pltpu.VMEM((CHUNK, D), jnp.float32),
            pltpu.VMEM((CHUNK,), jnp.int32),
            pltpu.VMEM((CHUNK,), jnp.int32),
            pltpu.VMEM((CHUNK, 16), jnp.float32),
            pltpu.VMEM((CHUNK, 16), jnp.float32),
            pltpu.SemaphoreType.DMA,
        ],
    )(_combine_body)


def kernel(x, Wr, br, W1, W2, W3):
    x2 = x.reshape(S, D)
    br2 = br.reshape(1, E)

    pos_col, w_splat, cnt_out = _run_router(x2, Wr, br2)
    pos = pos_col.reshape(NPAIR)

    cnt = cnt_out[0].astype(jnp.int32)
    tiles = (cnt + (T - 1)) // T
    tile_cum = jnp.cumsum(tiles)
    total = tile_cum[E - 1]
    j = jnp.arange(MAXTILES, dtype=jnp.int32)
    te_raw = jnp.sum(
        (j[:, None] >= tile_cum[None, :]).astype(jnp.int32), axis=1)
    last_e = jnp.max(jnp.where(tiles > 0, jnp.arange(E, dtype=jnp.int32), 0))
    te = jnp.where(j < total, te_raw, last_e).astype(jnp.int32)
    te = jnp.concatenate([te, total[None]]).astype(jnp.int32)

    xb = lax.bitcast_convert_type(
        x2.astype(jnp.bfloat16).reshape(S, D // 2, 2), jnp.int32)
    xs_i = _get_dispatch()(xb, pos)
    xs = lax.bitcast_convert_type(xs_i, jnp.bfloat16).reshape(NP, D)
    ys = _run_ffn(te, xs,
                  W1.astype(jnp.bfloat16),
                  W3.astype(jnp.bfloat16),
                  W2.astype(jnp.bfloat16))
    out = _get_combine()(ys, pos, w_splat)
    return out.reshape(1, S, D)


# speedup vs baseline: 1.4911x; 1.4911x over previous
import functools

import jax
import jax.numpy as jnp
from jax import lax
from jax.experimental import pallas as pl
from jax.experimental.pallas import tpu as pltpu
from jax.experimental.pallas import tpu_sc as plsc

S = 2048
D = 1024
F = 2816
E = 8
K = 2
T = 256
NPAIR = S * K
MAXTILES = NPAIR // T + (E - 1)
NP = MAXTILES * T

NC = 2
NS = 16
NW = NC * NS
TPW = S // NW
CHUNK = 32


def _router_body(x_ref, wr_ref, br_ref, pos_ref, w_ref, cnt_ref):
    x = x_ref[...]
    logits = jnp.dot(x, wr_ref[...], preferred_element_type=jnp.float32)
    logits = logits + br_ref[...]
    m = jnp.max(logits, axis=1, keepdims=True)
    ex = jnp.exp(logits - m)
    probs = ex / jnp.sum(ex, axis=1, keepdims=True)

    iota_e = lax.broadcasted_iota(jnp.int32, (S, E), 1)
    p1 = jnp.max(probs, axis=1, keepdims=True)
    i1 = jnp.min(jnp.where(probs == p1, iota_e, E), axis=1, keepdims=True)
    probs_m = jnp.where(iota_e == i1, -1.0, probs)
    p2 = jnp.max(probs_m, axis=1, keepdims=True)
    i2 = jnp.min(jnp.where(probs_m == p2, iota_e, E), axis=1, keepdims=True)
    ws = p1 + p2
    w0 = p1 / ws
    w1 = p2 / ws

    iota16 = lax.broadcasted_iota(jnp.int32, (S, 2 * E), 1)
    oh01 = jnp.where(iota16 < E, (i1 == iota16).astype(jnp.float32),
                     (i2 == (iota16 - E)).astype(jnp.float32))
    c = oh01
    sh = 1
    while sh < S:
        shifted = jnp.concatenate(
            [jnp.zeros((sh, 2 * E), jnp.float32), c[: S - sh, :]], axis=0)
        c = c + shifted
        sh *= 2
    tail = c[S - 1:S, :]
    cnt0 = tail[:, :E]
    cnt = cnt0 + tail[:, E:]
    tiles = jnp.floor((cnt + (T - 1)) * (1.0 / T))

    iota_e8 = lax.broadcasted_iota(jnp.int32, (S, E), 1)
    oh0 = (i1 == iota_e8).astype(jnp.float32)
    oh1 = (i2 == iota_e8).astype(jnp.float32)
    rank0 = jnp.sum(oh0 * c[:, :E], axis=1, keepdims=True) - 1.0
    rank1 = (jnp.sum(oh1 * c[:, E:], axis=1, keepdims=True) - 1.0
             + jnp.sum(oh1 * cnt0, axis=1, keepdims=True))
    seg0 = T * jnp.sum((iota_e8 < i1).astype(jnp.float32) * tiles,
                       axis=1, keepdims=True)
    seg1 = T * jnp.sum((iota_e8 < i2).astype(jnp.float32) * tiles,
                       axis=1, keepdims=True)
    pos_ref[:S, :] = (seg0 + rank0).astype(jnp.int32)
    pos_ref[S:, :] = (seg1 + rank1).astype(jnp.int32)
    wk = jnp.concatenate([w0, w1], axis=0)
    w_ref[...] = jnp.broadcast_to(wk, (NPAIR, 16))
    cnt_ref[...] = jnp.broadcast_to(cnt, (8, E))


def _run_router(x2, wr, br2):
    return pl.pallas_call(
        _router_body,
        out_shape=(
            jax.ShapeDtypeStruct((NPAIR, 1), jnp.int32),
            jax.ShapeDtypeStruct((NPAIR, 16), jnp.float32),
            jax.ShapeDtypeStruct((8, E), jnp.float32),
        ),
    )(x2, wr, br2)


def _dispatch_body(x_hbm, pos_hbm, xs_hbm, xbuf, idx0, idx1, sem):
    wid = lax.axis_index("s") * NC + lax.axis_index("c")
    base = wid * TPW
    pltpu.sync_copy(x_hbm.at[pl.ds(base, TPW)], xbuf)
    pltpu.sync_copy(pos_hbm.at[pl.ds(base, TPW)], idx0)
    pltpu.sync_copy(pos_hbm.at[pl.ds(S + base, TPW)], idx1)
    pltpu.async_copy(xbuf, xs_hbm.at[idx0], sem).wait()
    pltpu.async_copy(xbuf, xs_hbm.at[idx1], sem).wait()


@functools.cache
def _get_dispatch():
    return functools.partial(
        pl.kernel,
        out_type=jax.ShapeDtypeStruct((NP, D), jnp.float32),
        mesh=plsc.VectorSubcoreMesh(
            core_axis_name="c", subcore_axis_name="s",
            num_cores=NC, num_subcores=NS),
        scratch_types=[
            pltpu.VMEM((TPW, D), jnp.float32),
            pltpu.VMEM((TPW,), jnp.int32),
            pltpu.VMEM((TPW,), jnp.int32),
            pltpu.SemaphoreType.DMA,
        ],
    )(_dispatch_body)


def _ffn_body(te_ref, xs_ref, w1_ref, w3_ref, w2_ref, ys_ref):
    xt = xs_ref[...].astype(jnp.bfloat16)
    h1 = jnp.dot(xt, w1_ref[0], preferred_element_type=jnp.float32)
    h3 = jnp.dot(xt, w3_ref[0], preferred_element_type=jnp.float32)
    h = (h1 * jax.nn.sigmoid(h1) * h3).astype(jnp.bfloat16)
    ys_ref[...] = jnp.dot(h, w2_ref[0],
                          preferred_element_type=jnp.float32)


def _run_ffn(te, xs, w1, w3, w2):
    return pl.pallas_call(
        _ffn_body,
        grid_spec=pltpu.PrefetchScalarGridSpec(
            num_scalar_prefetch=1,
            grid=(MAXTILES,),
            in_specs=[
                pl.BlockSpec((T, D), lambda j, te: (j, 0)),
                pl.BlockSpec((1, D, F), lambda j, te: (te[j], 0, 0)),
                pl.BlockSpec((1, D, F), lambda j, te: (te[j], 0, 0)),
                pl.BlockSpec((1, F, D), lambda j, te: (te[j], 0, 0)),
            ],
            out_specs=pl.BlockSpec((T, D), lambda j, te: (j, 0)),
        ),
        out_shape=jax.ShapeDtypeStruct((NP, D), jnp.float32),
        name="moe_ffn",
    )(te, xs, w1, w3, w2)


def _combine_body(ys_hbm, pos_hbm, w_hbm, out_hbm,
                  buf0, buf1, obuf, idx0, idx1, wbuf0, wbuf1, sem):
    wid = lax.axis_index("s") * NC + lax.axis_index("c")
    for ci in range(TPW // CHUNK):
        base = wid * TPW + ci * CHUNK
        pltpu.sync_copy(pos_hbm.at[pl.ds(base, CHUNK)], idx0)
        pltpu.sync_copy(pos_hbm.at[pl.ds(S + base, CHUNK)], idx1)
        pltpu.sync_copy(w_hbm.at[pl.ds(base, CHUNK)], wbuf0)
        pltpu.sync_copy(w_hbm.at[pl.ds(S + base, CHUNK)], wbuf1)
        pltpu.async_copy(ys_hbm.at[idx0], buf0, sem).wait()
        pltpu.async_copy(ys_hbm.at[idx1], buf1, sem).wait()
        for i in range(CHUNK):
            wv0 = wbuf0[i, :]
            wv1 = wbuf1[i, :]

            def body(k, _, i=i, wv0=wv0, wv1=wv1):
                sl = pl.ds(k * 16, 16)
                obuf[i, sl] = wv0 * buf0[i, sl] + wv1 * buf1[i, sl]
                return 0

            lax.fori_loop(0, D // 16, body, 0, unroll=4)
        pltpu.sync_copy(obuf, out_hbm.at[pl.ds(base, CHUNK)])


@functools.cache
def _get_combine():
    return functools.partial(
        pl.kernel,
        out_type=jax.ShapeDtypeStruct((S, D), jnp.float32),
        mesh=plsc.VectorSubcoreMesh(
            core_axis_name="c", subcore_axis_name="s",
            num_cores=NC, num_subcores=NS),
        scratch_types=[
            pltpu.VMEM((CHUNK, D), jnp.float32),
            pltpu.VMEM((CHUNK, D), jnp.float32),
            pltpu.VMEM((CHUNK, D), jnp.float32),
            pltpu.VMEM((CHUNK,), jnp.int32),
            pltpu.VMEM((CHUNK,), jnp.int32),
            pltpu.VMEM((CHUNK, 16), jnp.float32),
            pltpu.VMEM((CHUNK, 16), jnp.float32),
            pltpu.SemaphoreType.DMA,
        ],
    )(_combine_body)


def kernel(x, Wr, br, W1, W2, W3):
    x2 = x.reshape(S, D)
    br2 = br.reshape(1, E)

    pos_col, w_splat, cnt_out = _run_router(x2, Wr, br2)
    pos = pos_col.reshape(NPAIR)

    cnt = cnt_out[0].astype(jnp.int32)
    tiles = (cnt + (T - 1)) // T
    tile_cum = jnp.cumsum(tiles)
    total = tile_cum[E - 1]
    j = jnp.arange(MAXTILES, dtype=jnp.int32)
    te_raw = jnp.sum(
        (j[:, None] >= tile_cum[None, :]).astype(jnp.int32), axis=1)
    last_e = jnp.max(jnp.where(tiles > 0, jnp.arange(E, dtype=jnp.int32), 0))
    te = jnp.where(j < total, te_raw, last_e).astype(jnp.int32)
    te = jnp.concatenate([te, total[None]]).astype(jnp.int32)

    xs = _get_dispatch()(x2, pos)
    ys = _run_ffn(te, xs,
                  W1.astype(jnp.bfloat16),
                  W3.astype(jnp.bfloat16),
                  W2.astype(jnp.bfloat16))
    out = _get_combine()(ys, pos, w_splat)
    return out.reshape(1, S, D)


# speedup vs baseline: 1.5207x; 1.0198x over previous
import functools

import jax
import jax.numpy as jnp
from jax import lax
from jax.experimental import pallas as pl
from jax.experimental.pallas import tpu as pltpu
from jax.experimental.pallas import tpu_sc as plsc

S = 2048
D = 1024
F = 2816
E = 8
K = 2
T = 256
NPAIR = S * K
MAXTILES = NPAIR // T + (E - 1)
NP = MAXTILES * T

NC = 2
NS = 16
NW = NC * NS
TPW = S // NW
CHUNK = 32


def _router_body(x_ref, wr_ref, br_ref, pos_ref, w_ref, cnt_ref):
    x = x_ref[...]
    logits = jnp.dot(x, wr_ref[...], preferred_element_type=jnp.float32)
    logits = logits + br_ref[...]
    m = jnp.max(logits, axis=1, keepdims=True)
    ex = jnp.exp(logits - m)
    probs = ex / jnp.sum(ex, axis=1, keepdims=True)

    iota_e = lax.broadcasted_iota(jnp.int32, (S, E), 1)
    p1 = jnp.max(probs, axis=1, keepdims=True)
    i1 = jnp.min(jnp.where(probs == p1, iota_e, E), axis=1, keepdims=True)
    probs_m = jnp.where(iota_e == i1, -1.0, probs)
    p2 = jnp.max(probs_m, axis=1, keepdims=True)
    i2 = jnp.min(jnp.where(probs_m == p2, iota_e, E), axis=1, keepdims=True)
    ws = p1 + p2
    w0 = p1 / ws
    w1 = p2 / ws

    iota16 = lax.broadcasted_iota(jnp.int32, (S, 2 * E), 1)
    oh01 = jnp.where(iota16 < E, (i1 == iota16).astype(jnp.float32),
                     (i2 == (iota16 - E)).astype(jnp.float32))
    c = oh01
    sh = 1
    while sh < S:
        shifted = jnp.concatenate(
            [jnp.zeros((sh, 2 * E), jnp.float32), c[: S - sh, :]], axis=0)
        c = c + shifted
        sh *= 2
    tail = c[S - 1:S, :]
    cnt0 = tail[:, :E]
    cnt = cnt0 + tail[:, E:]
    tiles = jnp.floor((cnt + (T - 1)) * (1.0 / T))

    iota_e8 = lax.broadcasted_iota(jnp.int32, (S, E), 1)
    oh0 = (i1 == iota_e8).astype(jnp.float32)
    oh1 = (i2 == iota_e8).astype(jnp.float32)
    rank0 = jnp.sum(oh0 * c[:, :E], axis=1, keepdims=True) - 1.0
    rank1 = (jnp.sum(oh1 * c[:, E:], axis=1, keepdims=True) - 1.0
             + jnp.sum(oh1 * cnt0, axis=1, keepdims=True))
    seg0 = T * jnp.sum((iota_e8 < i1).astype(jnp.float32) * tiles,
                       axis=1, keepdims=True)
    seg1 = T * jnp.sum((iota_e8 < i2).astype(jnp.float32) * tiles,
                       axis=1, keepdims=True)
    pos_ref[:S, :] = (seg0 + rank0).astype(jnp.int32)
    pos_ref[S:, :] = (seg1 + rank1).astype(jnp.int32)
    wk = jnp.concatenate([w0, w1], axis=0)
    w_ref[...] = jnp.broadcast_to(wk, (NPAIR, 16))
    cnt_ref[...] = jnp.broadcast_to(cnt, (8, E))


def _run_router(x2, wr, br2):
    return pl.pallas_call(
        _router_body,
        out_shape=(
            jax.ShapeDtypeStruct((NPAIR, 1), jnp.int32),
            jax.ShapeDtypeStruct((NPAIR, 16), jnp.float32),
            jax.ShapeDtypeStruct((8, E), jnp.float32),
        ),
    )(x2, wr, br2)


def _dispatch_body(x_hbm, pos_hbm, xs_hbm, xbuf, idx0, idx1, sem):
    wid = lax.axis_index("s") * NC + lax.axis_index("c")
    base = wid * TPW
    pltpu.sync_copy(x_hbm.at[pl.ds(base, TPW)], xbuf)
    pltpu.sync_copy(pos_hbm.at[pl.ds(base, TPW)], idx0)
    pltpu.sync_copy(pos_hbm.at[pl.ds(S + base, TPW)], idx1)
    pltpu.async_copy(xbuf, xs_hbm.at[idx0], sem).wait()
    pltpu.async_copy(xbuf, xs_hbm.at[idx1], sem).wait()


@functools.cache
def _get_dispatch():
    return functools.partial(
        pl.kernel,
        out_type=jax.ShapeDtypeStruct((NP, D), jnp.float32),
        mesh=plsc.VectorSubcoreMesh(
            core_axis_name="c", subcore_axis_name="s",
            num_cores=NC, num_subcores=NS),
        scratch_types=[
            pltpu.VMEM((TPW, D), jnp.float32),
            pltpu.VMEM((TPW,), jnp.int32),
            pltpu.VMEM((TPW,), jnp.int32),
            pltpu.SemaphoreType.DMA,
        ],
    )(_dispatch_body)


def _ffn_body(te_ref, xs_ref, w1_ref, w3_ref, w2_ref, ys_ref):
    @pl.when(pl.program_id(0) < te_ref[MAXTILES])
    def _():
        xt = xs_ref[...].astype(jnp.bfloat16)
        h1 = jnp.dot(xt, w1_ref[0], preferred_element_type=jnp.float32)
        h3 = jnp.dot(xt, w3_ref[0], preferred_element_type=jnp.float32)
        h = (h1 * jax.nn.sigmoid(h1) * h3).astype(jnp.bfloat16)
        ys_ref[...] = jnp.dot(h, w2_ref[0],
                              preferred_element_type=jnp.float32)


def _run_ffn(te, xs, w1, w3, w2):
    return pl.pallas_call(
        _ffn_body,
        grid_spec=pltpu.PrefetchScalarGridSpec(
            num_scalar_prefetch=1,
            grid=(MAXTILES,),
            in_specs=[
                pl.BlockSpec((T, D), lambda j, te: (j, 0)),
                pl.BlockSpec((1, D, F), lambda j, te: (te[j], 0, 0)),
                pl.BlockSpec((1, D, F), lambda j, te: (te[j], 0, 0)),
                pl.BlockSpec((1, F, D), lambda j, te: (te[j], 0, 0)),
            ],
            out_specs=pl.BlockSpec((T, D), lambda j, te: (j, 0)),
        ),
        out_shape=jax.ShapeDtypeStruct((NP, D), jnp.float32),
        name="moe_ffn",
    )(te, xs, w1, w3, w2)


def _combine_body(ys_hbm, pos_hbm, w_hbm, out_hbm,
                  buf0, buf1, obuf, idx0, idx1, wbuf0, wbuf1, sem):
    wid = lax.axis_index("s") * NC + lax.axis_index("c")
    for ci in range(TPW // CHUNK):
        base = wid * TPW + ci * CHUNK
        pltpu.sync_copy(pos_hbm.at[pl.ds(base, CHUNK)], idx0)
        pltpu.sync_copy(pos_hbm.at[pl.ds(S + base, CHUNK)], idx1)
        pltpu.sync_copy(w_hbm.at[pl.ds(base, CHUNK)], wbuf0)
        pltpu.sync_copy(w_hbm.at[pl.ds(S + base, CHUNK)], wbuf1)
        pltpu.async_copy(ys_hbm.at[idx0], buf0, sem).wait()
        pltpu.async_copy(ys_hbm.at[idx1], buf1, sem).wait()
        for i in range(CHUNK):
            wv0 = wbuf0[i, :]
            wv1 = wbuf1[i, :]

            def body(k, _, i=i, wv0=wv0, wv1=wv1):
                sl = pl.ds(k * 16, 16)
                obuf[i, sl] = wv0 * buf0[i, sl] + wv1 * buf1[i, sl]
                return 0

            lax.fori_loop(0, D // 16, body, 0, unroll=4)
        pltpu.sync_copy(obuf, out_hbm.at[pl.ds(base, CHUNK)])


@functools.cache
def _get_combine():
    return functools.partial(
        pl.kernel,
        out_type=jax.ShapeDtypeStruct((S, D), jnp.float32),
        mesh=plsc.VectorSubcoreMesh(
            core_axis_name="c", subcore_axis_name="s",
            num_cores=NC, num_subcores=NS),
        scratch_types=[
            pltpu.VMEM((CHUNK, D), jnp.float32),
            pltpu.VMEM((CHUNK, D), jnp.float32),
            pltpu.VMEM((CHUNK, D), jnp.float32),
            pltpu.VMEM((CHUNK,), jnp.int32),
            pltpu.VMEM((CHUNK,), jnp.int32),
            pltpu.VMEM((CHUNK, 16), jnp.float32),
            pltpu.VMEM((CHUNK, 16), jnp.float32),
            pltpu.SemaphoreType.DMA,
        ],
    )(_combine_body)


def kernel(x, Wr, br, W1, W2, W3):
    x2 = x.reshape(S, D)
    br2 = br.reshape(1, E)

    pos_col, w_splat, cnt_out = _run_router(x2, Wr, br2)
    pos = pos_col.reshape(NPAIR)

    cnt = cnt_out[0].astype(jnp.int32)
    tiles = (cnt + (T - 1)) // T
    tile_cum = jnp.cumsum(tiles)
    total = tile_cum[E - 1]
    j = jnp.arange(MAXTILES, dtype=jnp.int32)
    te_raw = jnp.sum(
        (j[:, None] >= tile_cum[None, :]).astype(jnp.int32), axis=1)
    last_e = jnp.max(jnp.where(tiles > 0, jnp.arange(E, dtype=jnp.int32), 0))
    te = jnp.where(j < total, te_raw, last_e).astype(jnp.int32)
    te = jnp.concatenate([te, total[None]]).astype(jnp.int32)

    xs = _get_dispatch()(x2, pos)
    ys = _run_ffn(te, xs,
                  W1.astype(jnp.bfloat16),
                  W3.astype(jnp.bfloat16),
                  W2.astype(jnp.bfloat16))
    out = _get_combine()(ys, pos, w_splat)
    return out.reshape(1, S, D)


# speedup vs baseline: 1.5230x; 1.0015x over previous
import functools

import jax
import jax.numpy as jnp
from jax import lax
from jax.experimental import pallas as pl
from jax.experimental.pallas import tpu as pltpu
from jax.experimental.pallas import tpu_sc as plsc

S = 2048
D = 1024
F = 2816
E = 8
K = 2
T = 256
NPAIR = S * K
MAXTILES = NPAIR // T + (E - 1)
NP = MAXTILES * T

NC = 2
NS = 16
NW = NC * NS
TPW = S // NW
CHUNK = 32


def _router_body(x_ref, wr_ref, br_ref, pos_ref, w_ref, cnt_ref):
    x = x_ref[...]
    logits = jnp.dot(x, wr_ref[...], preferred_element_type=jnp.float32)
    logits = logits + br_ref[...]
    m = jnp.max(logits, axis=1, keepdims=True)
    ex = jnp.exp(logits - m)
    probs = ex / jnp.sum(ex, axis=1, keepdims=True)

    iota_e = lax.broadcasted_iota(jnp.int32, (S, E), 1)
    p1 = jnp.max(probs, axis=1, keepdims=True)
    i1 = jnp.min(jnp.where(probs == p1, iota_e, E), axis=1, keepdims=True)
    probs_m = jnp.where(iota_e == i1, -1.0, probs)
    p2 = jnp.max(probs_m, axis=1, keepdims=True)
    i2 = jnp.min(jnp.where(probs_m == p2, iota_e, E), axis=1, keepdims=True)
    ws = p1 + p2
    w0 = p1 / ws
    w1 = p2 / ws

    iota16 = lax.broadcasted_iota(jnp.int32, (S, 2 * E), 1)
    oh01 = jnp.where(iota16 < E, (i1 == iota16).astype(jnp.float32),
                     (i2 == (iota16 - E)).astype(jnp.float32))
    c = oh01
    sh = 1
    while sh < S:
        shifted = jnp.concatenate(
            [jnp.zeros((sh, 2 * E), jnp.float32), c[: S - sh, :]], axis=0)
        c = c + shifted
        sh *= 2
    tail = c[S - 1:S, :]
    cnt0 = tail[:, :E]
    cnt = cnt0 + tail[:, E:]
    tiles = jnp.floor((cnt + (T - 1)) * (1.0 / T))

    iota_e8 = lax.broadcasted_iota(jnp.int32, (S, E), 1)
    oh0 = (i1 == iota_e8).astype(jnp.float32)
    oh1 = (i2 == iota_e8).astype(jnp.float32)
    rank0 = jnp.sum(oh0 * c[:, :E], axis=1, keepdims=True) - 1.0
    rank1 = (jnp.sum(oh1 * c[:, E:], axis=1, keepdims=True) - 1.0
             + jnp.sum(oh1 * cnt0, axis=1, keepdims=True))
    seg0 = T * jnp.sum((iota_e8 < i1).astype(jnp.float32) * tiles,
                       axis=1, keepdims=True)
    seg1 = T * jnp.sum((iota_e8 < i2).astype(jnp.float32) * tiles,
                       axis=1, keepdims=True)
    pos_ref[:S, :] = (seg0 + rank0).astype(jnp.int32)
    pos_ref[S:, :] = (seg1 + rank1).astype(jnp.int32)
    wk = jnp.concatenate([w0, w1], axis=0)
    w_ref[...] = jnp.broadcast_to(wk, (NPAIR, 16))
    cnt_ref[...] = jnp.broadcast_to(cnt, (8, E))


def _run_router(x2, wr, br2):
    return pl.pallas_call(
        _router_body,
        out_shape=(
            jax.ShapeDtypeStruct((NPAIR, 1), jnp.int32),
            jax.ShapeDtypeStruct((NPAIR, 16), jnp.float32),
            jax.ShapeDtypeStruct((8, E), jnp.float32),
        ),
    )(x2, wr, br2)


def _dispatch_body(x_hbm, pos_hbm, xs_hbm, xbuf, idx0, idx1, sem):
    wid = lax.axis_index("s") * NC + lax.axis_index("c")
    base = wid * TPW
    pltpu.sync_copy(x_hbm.at[pl.ds(base, TPW)], xbuf)
    pltpu.sync_copy(pos_hbm.at[pl.ds(base, TPW)], idx0)
    pltpu.sync_copy(pos_hbm.at[pl.ds(S + base, TPW)], idx1)
    c0 = pltpu.async_copy(xbuf, xs_hbm.at[idx0], sem)
    c1 = pltpu.async_copy(xbuf, xs_hbm.at[idx1], sem)
    c0.wait()
    c1.wait()


@functools.cache
def _get_dispatch():
    return functools.partial(
        pl.kernel,
        out_type=jax.ShapeDtypeStruct((NP, D), jnp.float32),
        mesh=plsc.VectorSubcoreMesh(
            core_axis_name="c", subcore_axis_name="s",
            num_cores=NC, num_subcores=NS),
        scratch_types=[
            pltpu.VMEM((TPW, D), jnp.float32),
            pltpu.VMEM((TPW,), jnp.int32),
            pltpu.VMEM((TPW,), jnp.int32),
            pltpu.SemaphoreType.DMA,
        ],
    )(_dispatch_body)


def _ffn_body(te_ref, xs_ref, w1_ref, w3_ref, w2_ref, ys_ref):
    @pl.when(pl.program_id(0) < te_ref[MAXTILES])
    def _():
        xt = xs_ref[...].astype(jnp.bfloat16)
        h1 = jnp.dot(xt, w1_ref[0], preferred_element_type=jnp.float32)
        h3 = jnp.dot(xt, w3_ref[0], preferred_element_type=jnp.float32)
        h = (h1 * jax.nn.sigmoid(h1) * h3).astype(jnp.bfloat16)
        ys_ref[...] = jnp.dot(h, w2_ref[0],
                              preferred_element_type=jnp.float32)


def _run_ffn(te, xs, w1, w3, w2):
    return pl.pallas_call(
        _ffn_body,
        grid_spec=pltpu.PrefetchScalarGridSpec(
            num_scalar_prefetch=1,
            grid=(MAXTILES,),
            in_specs=[
                pl.BlockSpec((T, D), lambda j, te: (j, 0)),
                pl.BlockSpec((1, D, F), lambda j, te: (te[j], 0, 0)),
                pl.BlockSpec((1, D, F), lambda j, te: (te[j], 0, 0)),
                pl.BlockSpec((1, F, D), lambda j, te: (te[j], 0, 0)),
            ],
            out_specs=pl.BlockSpec((T, D), lambda j, te: (j, 0)),
        ),
        out_shape=jax.ShapeDtypeStruct((NP, D), jnp.float32),
        name="moe_ffn",
    )(te, xs, w1, w3, w2)


def _combine_body(ys_hbm, pos_hbm, w_hbm, out_hbm,
                  buf0, buf1, obuf, idx0, idx1, wbuf0, wbuf1, sem):
    wid = lax.axis_index("s") * NC + lax.axis_index("c")
    for ci in range(TPW // CHUNK):
        base = wid * TPW + ci * CHUNK
        pltpu.sync_copy(pos_hbm.at[pl.ds(base, CHUNK)], idx0)
        pltpu.sync_copy(pos_hbm.at[pl.ds(S + base, CHUNK)], idx1)
        pltpu.sync_copy(w_hbm.at[pl.ds(base, CHUNK)], wbuf0)
        pltpu.sync_copy(w_hbm.at[pl.ds(S + base, CHUNK)], wbuf1)
        g0 = pltpu.async_copy(ys_hbm.at[idx0], buf0, sem)
        g1 = pltpu.async_copy(ys_hbm.at[idx1], buf1, sem)
        g0.wait()
        g1.wait()
        for i in range(CHUNK):
            wv0 = wbuf0[i, :]
            wv1 = wbuf1[i, :]

            def body(k, _, i=i, wv0=wv0, wv1=wv1):
                sl = pl.ds(k * 16, 16)
                obuf[i, sl] = wv0 * buf0[i, sl] + wv1 * buf1[i, sl]
                return 0

            lax.fori_loop(0, D // 16, body, 0, unroll=4)
        pltpu.sync_copy(obuf, out_hbm.at[pl.ds(base, CHUNK)])


@functools.cache
def _get_combine():
    return functools.partial(
        pl.kernel,
        out_type=jax.ShapeDtypeStruct((S, D), jnp.float32),
        mesh=plsc.VectorSubcoreMesh(
            core_axis_name="c", subcore_axis_name="s",
            num_cores=NC, num_subcores=NS),
        scratch_types=[
            pltpu.VMEM((CHUNK, D), jnp.float32),
            pltpu.VMEM((CHUNK, D), jnp.float32),
            pltpu.VMEM((CHUNK, D), jnp.float32),
            pltpu.VMEM((CHUNK,), jnp.int32),
            pltpu.VMEM((CHUNK,), jnp.int32),
            pltpu.VMEM((CHUNK, 16), jnp.float32),
            pltpu.VMEM((CHUNK, 16), jnp.float32),
            pltpu.SemaphoreType.DMA,
        ],
    )(_combine_body)


def kernel(x, Wr, br, W1, W2, W3):
    x2 = x.reshape(S, D)
    br2 = br.reshape(1, E)

    pos_col, w_splat, cnt_out = _run_router(x2, Wr, br2)
    pos = pos_col.reshape(NPAIR)

    cnt = cnt_out[0].astype(jnp.int32)
    tiles = (cnt + (T - 1)) // T
    tile_cum = jnp.cumsum(tiles)
    total = tile_cum[E - 1]
    j = jnp.arange(MAXTILES, dtype=jnp.int32)
    te_raw = jnp.sum(
        (j[:, None] >= tile_cum[None, :]).astype(jnp.int32), axis=1)
    last_e = jnp.max(jnp.where(tiles > 0, jnp.arange(E, dtype=jnp.int32), 0))
    te = jnp.where(j < total, te_raw, last_e).astype(jnp.int32)
    te = jnp.concatenate([te, total[None]]).astype(jnp.int32)

    xs = _get_dispatch()(x2, pos)
    ys = _run_ffn(te, xs,
                  W1.astype(jnp.bfloat16),
                  W3.astype(jnp.bfloat16),
                  W2.astype(jnp.bfloat16))
    out = _get_combine()(ys, pos, w_splat)
    return out.reshape(1, S, D)
